# baseline (speedup 1.0000x reference)
import jax
import jax.numpy as jnp
from jax.experimental import pallas as pl
from jax.experimental.pallas import tpu as pltpu

_B = 128
_R = 768
_C = 256


def _fma_body(ts_ref, ta_ref, tb_ref, o_ref, n_ref, out_ref):
    b = pl.program_id(0)
    t = ts_ref[b]
    a = ta_ref[t]
    c = tb_ref[t]
    out_ref[...] = a * o_ref[...] + c * n_ref[...]


def kernel(original_samples, noise, timesteps, sqrt_alphas_cumprod,
           sqrt_one_minus_alphas_cumprod):
    orig = original_samples.reshape(_B, _R, _C)
    nz = noise.reshape(_B, _R, _C)
    ts = timesteps.astype(jnp.int32)

    grid_spec = pltpu.PrefetchScalarGridSpec(
        num_scalar_prefetch=3,
        grid=(_B,),
        in_specs=[
            pl.BlockSpec((1, _R, _C), lambda i, *_: (i, 0, 0)),
            pl.BlockSpec((1, _R, _C), lambda i, *_: (i, 0, 0)),
        ],
        out_specs=pl.BlockSpec((1, _R, _C), lambda i, *_: (i, 0, 0)),
    )

    out = pl.pallas_call(
        _fma_body,
        grid_spec=grid_spec,
        out_shape=jax.ShapeDtypeStruct((_B, _R, _C), jnp.float32),
    )(ts, sqrt_alphas_cumprod, sqrt_one_minus_alphas_cumprod, orig, nz)

    return out.reshape(original_samples.shape)


# speedup vs baseline: 1.4187x; 1.4187x over previous
import jax
import jax.numpy as jnp
from jax.experimental import pallas as pl
from jax.experimental.pallas import tpu as pltpu

_B = 128
_R = 768
_C = 256


_BB = 4


def _fma_body(ts_ref, ta_ref, tb_ref, o_ref, n_ref, out_ref):
    i = pl.program_id(0)
    for j in range(_BB):
        t = ts_ref[i * _BB + j]
        a = ta_ref[t]
        c = tb_ref[t]
        out_ref[j] = a * o_ref[j] + c * n_ref[j]


def kernel(original_samples, noise, timesteps, sqrt_alphas_cumprod,
           sqrt_one_minus_alphas_cumprod):
    orig = original_samples.reshape(_B, _R, _C)
    nz = noise.reshape(_B, _R, _C)
    ts = timesteps.astype(jnp.int32)

    grid_spec = pltpu.PrefetchScalarGridSpec(
        num_scalar_prefetch=3,
        grid=(_B // _BB,),
        in_specs=[
            pl.BlockSpec((_BB, _R, _C), lambda i, *_: (i, 0, 0)),
            pl.BlockSpec((_BB, _R, _C), lambda i, *_: (i, 0, 0)),
        ],
        out_specs=pl.BlockSpec((_BB, _R, _C), lambda i, *_: (i, 0, 0)),
    )

    out = pl.pallas_call(
        _fma_body,
        grid_spec=grid_spec,
        out_shape=jax.ShapeDtypeStruct((_B, _R, _C), jnp.float32),
        compiler_params=pltpu.CompilerParams(
            dimension_semantics=("arbitrary",)),
    )(ts, sqrt_alphas_cumprod, sqrt_one_minus_alphas_cumprod, orig, nz)

    return out.reshape(original_samples.shape)
